# baseline (speedup 1.0000x reference)
import functools

import jax
import jax.numpy as jnp
import numpy as np
from jax import lax
from jax.experimental import pallas as pl
from jax.experimental.pallas import tpu as pltpu
from jax.experimental.pallas import tpu_sc as plsc

B = 2
C = 80
NPRIME = 506880
PPB = NPRIME // B
NS = 16
PPT = PPB // NS
CH = 16
NPASS = C // CH
GRID = 200
CELLS = GRID * GRID
DUMP = CELLS
ACC_ROWS = CELLS + 8
CHUNK = 96
NCHUNK = PPT // CHUNK
CPS = 11
SLAB = CHUNK * CPS
NSLAB = PPT // SLAB
GCH = 3168
NGCH = PPT // GCH
GCC = GCH // CHUNK
DRAIN = 2560
DHALF = DRAIN // 2
DSTRIDE = 2496
ZROWS = 256

_DX = np.float32(0.005)
_DZ = np.float32(1.0)

_mesh = plsc.VectorSubcoreMesh(core_axis_name="c", subcore_axis_name="s")


@functools.partial(
    pl.kernel,
    mesh=_mesh,
    compiler_params=pltpu.CompilerParams(
        use_tc_tiling_on_sc=False, needs_layout_passes=False),
    out_type=jax.ShapeDtypeStruct((B, C, CELLS), jnp.float32),
    scratch_types=[
        pltpu.VMEM((GCH * 3,), jnp.float32),
        pltpu.VMEM((NCHUNK, CHUNK), jnp.int32),
        pltpu.VMEM((SLAB, CH), jnp.float32),
        pltpu.VMEM((DHALF, CH), jnp.float32),
        pltpu.VMEM((CH, DHALF), jnp.float32),
        pltpu.VMEM((ZROWS, CH), jnp.float32),
        pltpu.VMEM_SHARED((ACC_ROWS, CH), jnp.float32),
    ],
)
def _bevpool_sc(g_hbm, x_hbm, out_hbm, gbuf, idxbuf, xbuf, tbuf, tbuf2,
                zbuf, acc):
    c = lax.axis_index("c")
    s = lax.axis_index("s")
    gbase = c * PPB + s * PPT
    iota = lax.iota(jnp.int32, 16)
    zvec = jnp.zeros((16,), jnp.float32)

    def _fill_zero(i, carry):
        zbuf[i, :] = zvec
        return carry

    lax.fori_loop(0, ZROWS, _fill_zero, 0)

    for g in range(NGCH):
        pltpu.sync_copy(
            g_hbm.at[pl.ds((gbase + g * GCH) * 3, GCH * 3)], gbuf)

        def _voxelize(r, carry, g=g):
            for cc in range(CHUNK // 16):
                o = (r * (CHUNK // 16) + cc) * 16
                pi = (o + iota) * 3
                vx = plsc.load_gather(gbuf, [pi])
                vy = plsc.load_gather(gbuf, [pi + 1])
                vz = plsc.load_gather(gbuf, [pi + 2])
                ix = (vx / _DX).astype(jnp.int32)
                iy = (vy / _DX).astype(jnp.int32)
                iz = (vz / _DZ).astype(jnp.int32)
                kept = (
                    (ix >= 0) & (ix < GRID)
                    & (iy >= 0) & (iy < GRID)
                    & (iz >= 0) & (iz < 1)
                )
                lin = ix * GRID + iy
                idxbuf[g * GCC + r, pl.ds(cc * 16, 16)] = jnp.where(
                    kept, lin, DUMP)
            return carry

        lax.fori_loop(0, GCC, _voxelize, 0)

    roff = jnp.minimum(s * DSTRIDE, CELLS - DRAIN)

    for p in range(NPASS):
        for z in range(DRAIN // ZROWS):
            pltpu.sync_copy(zbuf, acc.at[pl.ds(roff + z * ZROWS, ZROWS)])
        plsc.subcore_barrier()

        def _slab(t, carry, p=p):
            pltpu.sync_copy(
                x_hbm.at[pl.ds(gbase + t * SLAB, SLAB), pl.ds(p * CH, CH)],
                xbuf,
            )
            for j in range(CPS):
                pltpu.sync_copy(
                    xbuf.at[pl.ds(j * CHUNK, CHUNK)],
                    acc.at[idxbuf.at[t * CPS + j]],
                    add=True,
                )
            return carry

        lax.fori_loop(0, NSLAB, _slab, 0)
        plsc.subcore_barrier()

        for h in range(2):
            pltpu.sync_copy(acc.at[pl.ds(roff + h * DHALF, DHALF)], tbuf)

            def _transpose(b, carry):
                row = b * 16 + iota
                for cc in range(CH):
                    col = jnp.full((16,), cc, jnp.int32)
                    tbuf2[cc, pl.ds(b * 16, 16)] = plsc.load_gather(
                        tbuf, [row, col])
                return carry

            lax.fori_loop(0, DHALF // 16, _transpose, 0)
            pltpu.sync_copy(
                tbuf2,
                out_hbm.at[c, pl.ds(p * CH, CH),
                           pl.ds(roff + h * DHALF, DHALF)],
            )
        plsc.subcore_barrier()


def kernel(geom_feats, x):
    g = geom_feats.reshape(NPRIME * 3)
    x2d = x.reshape(NPRIME, C)
    out = _bevpool_sc(g, x2d)
    return out.reshape(B, C, GRID, GRID)


# speedup vs baseline: 1.0313x; 1.0313x over previous
import functools

import jax
import jax.numpy as jnp
import numpy as np
from jax import lax
from jax.experimental import pallas as pl
from jax.experimental.pallas import tpu as pltpu
from jax.experimental.pallas import tpu_sc as plsc

B = 2
C = 80
NPRIME = 506880
PPB = NPRIME // B
NS = 16
PPT = PPB // NS
CH = 16
NPASS = C // CH
GRID = 200
CELLS = GRID * GRID
DUMP = CELLS
ACC_ROWS = CELLS + 8
CHUNK = 96
NCHUNK = PPT // CHUNK
CPS = 5
SLAB = CHUNK * CPS
NBUF = 3
NGRP = PPT // (SLAB * NBUF)
GCH = 3168
NGCH = PPT // GCH
GCC = GCH // CHUNK
DRAIN = 2560
DHALF = DRAIN // 4
DSTRIDE = 2496
ZROWS = 256

_DX = np.float32(0.005)
_DZ = np.float32(1.0)

_mesh = plsc.VectorSubcoreMesh(core_axis_name="c", subcore_axis_name="s")


@functools.partial(
    pl.kernel,
    mesh=_mesh,
    compiler_params=pltpu.CompilerParams(
        use_tc_tiling_on_sc=False, needs_layout_passes=False),
    out_type=jax.ShapeDtypeStruct((B, C, CELLS), jnp.float32),
    scratch_types=[
        pltpu.VMEM((GCH * 3,), jnp.float32),
        pltpu.VMEM((NCHUNK, CHUNK), jnp.int32),
        pltpu.VMEM((NBUF, SLAB, CH), jnp.float32),
        pltpu.VMEM((DHALF, CH + 1), jnp.float32),
        pltpu.VMEM((CH, DHALF), jnp.float32),
        pltpu.VMEM((ZROWS, CH), jnp.float32),
        pltpu.VMEM_SHARED((ACC_ROWS, CH), jnp.float32),
        pltpu.SemaphoreType.DMA,
        pltpu.SemaphoreType.DMA,
        pltpu.SemaphoreType.DMA,
        pltpu.SemaphoreType.DMA,
    ],
)
def _bevpool_sc(g_hbm, x_hbm, out_hbm, gbuf, idxbuf, xbuf, tbuf, tbuf2,
                zbuf, acc, lsem0, lsem1, lsem2, ssem):
    lsems = (lsem0, lsem1, lsem2)
    c = lax.axis_index("c")
    s = lax.axis_index("s")
    gbase = c * PPB + s * PPT
    iota = lax.iota(jnp.int32, 16)
    zvec = jnp.zeros((16,), jnp.float32)

    def _fill_zero(i, carry):
        zbuf[i, :] = zvec
        return carry

    lax.fori_loop(0, ZROWS, _fill_zero, 0)

    for g in range(NGCH):
        pltpu.sync_copy(
            g_hbm.at[pl.ds((gbase + g * GCH) * 3, GCH * 3)], gbuf)

        def _voxelize(r, carry, g=g):
            for cc in range(CHUNK // 16):
                o = (r * (CHUNK // 16) + cc) * 16
                pi = (o + iota) * 3
                vx = plsc.load_gather(gbuf, [pi])
                vy = plsc.load_gather(gbuf, [pi + 1])
                vz = plsc.load_gather(gbuf, [pi + 2])
                ix = (vx / _DX).astype(jnp.int32)
                iy = (vy / _DX).astype(jnp.int32)
                iz = (vz / _DZ).astype(jnp.int32)
                kept = (
                    (ix >= 0) & (ix < GRID)
                    & (iy >= 0) & (iy < GRID)
                    & (iz >= 0) & (iz < 1)
                )
                lin = ix * GRID + iy
                idxbuf[g * GCC + r, pl.ds(cc * 16, 16)] = jnp.where(
                    kept, lin, DUMP)
            return carry

        lax.fori_loop(0, GCC, _voxelize, 0)

    roff = jnp.minimum(s * DSTRIDE, CELLS - DRAIN)

    for p in range(NPASS):
        for z in range(DRAIN // ZROWS):
            pltpu.sync_copy(zbuf, acc.at[pl.ds(roff + z * ZROWS, ZROWS)])
        plsc.subcore_barrier()

        def _group(m, carry, p=p):
            t0 = m * NBUF
            loads = []
            for k in range(NBUF):
                loads.append(pltpu.async_copy(
                    x_hbm.at[pl.ds(gbase + (t0 + k) * SLAB, SLAB),
                             pl.ds(p * CH, CH)],
                    xbuf.at[k], lsems[k]))
            scats = []
            for k in range(NBUF):
                loads[k].wait()
                for j in range(CPS):
                    scats.append(pltpu.async_copy(
                        xbuf.at[k, pl.ds(j * CHUNK, CHUNK)],
                        acc.at[idxbuf.at[(t0 + k) * CPS + j]],
                        ssem, add=True))
            for h in scats:
                h.wait()
            return carry

        lax.fori_loop(0, NGRP, _group, 0)
        plsc.subcore_barrier()

        for h in range(DRAIN // DHALF):
            pltpu.sync_copy(acc.at[pl.ds(roff + h * DHALF, DHALF)],
                            tbuf.at[:, pl.ds(0, CH)])

            def _transpose(b, carry):
                row = b * 16 + iota
                for cc in range(CH):
                    col = jnp.full((16,), cc, jnp.int32)
                    tbuf2[cc, pl.ds(b * 16, 16)] = plsc.load_gather(
                        tbuf, [row, col])
                return carry

            lax.fori_loop(0, DHALF // 16, _transpose, 0)
            pltpu.sync_copy(
                tbuf2,
                out_hbm.at[c, pl.ds(p * CH, CH),
                           pl.ds(roff + h * DHALF, DHALF)],
            )
        plsc.subcore_barrier()


def kernel(geom_feats, x):
    g = geom_feats.reshape(NPRIME * 3)
    x2d = x.reshape(NPRIME, C)
    out = _bevpool_sc(g, x2d)
    return out.reshape(B, C, GRID, GRID)


# speedup vs baseline: 2.6088x; 2.5296x over previous
import functools

import jax
import jax.numpy as jnp
import numpy as np
from jax import lax
from jax.experimental import pallas as pl
from jax.experimental.pallas import tpu as pltpu
from jax.experimental.pallas import tpu_sc as plsc

B = 2
C = 80
NPRIME = 506880
PPB = NPRIME // B
NS = 16
PPT = PPB // NS
CH = 16
NPASS = C // CH
GRID = 200
CELLS = GRID * GRID
DUMP = CELLS
ACC_ROWS = CELLS + 8
CHUNK = 96
NCHUNK = PPT // CHUNK
CPS = 5
SLAB = CHUNK * CPS
NBUF = 3
NGRP = PPT // (SLAB * NBUF)
ROWS_PER_TILE = CELLS // NS
ZROWS = 250

_DX = np.float32(0.005)
_DZ = np.float32(1.0)

_mesh = plsc.VectorSubcoreMesh(core_axis_name="c", subcore_axis_name="s")


@functools.partial(
    pl.kernel,
    mesh=_mesh,
    compiler_params=pltpu.CompilerParams(use_tc_tiling_on_sc=False),
    out_type=jax.ShapeDtypeStruct((B, NPASS, CELLS, CH), jnp.float32),
    scratch_types=[
        pltpu.VMEM((3, PPT), jnp.float32),
        pltpu.VMEM((NCHUNK, CHUNK), jnp.int32),
        pltpu.VMEM((NBUF, SLAB, CH), jnp.float32),
        pltpu.VMEM((ZROWS, CH), jnp.float32),
        pltpu.VMEM_SHARED((ACC_ROWS, CH), jnp.float32),
        pltpu.SemaphoreType.DMA,
        pltpu.SemaphoreType.DMA,
        pltpu.SemaphoreType.DMA,
        pltpu.SemaphoreType.DMA,
    ],
)
def _bevpool_sc(g_hbm, x_hbm, out_hbm, gbuf, idxbuf, xbuf,
                zbuf, acc, lsem0, lsem1, lsem2, ssem):
    lsems = (lsem0, lsem1, lsem2)
    c = lax.axis_index("c")
    s = lax.axis_index("s")
    gbase = c * PPB + s * PPT
    zvec = jnp.zeros((16,), jnp.float32)

    def _fill_zero(i, carry):
        zbuf[i, :] = zvec
        return carry

    lax.fori_loop(0, ZROWS, _fill_zero, 0)

    pltpu.sync_copy(g_hbm.at[:, pl.ds(gbase, PPT)], gbuf)

    def _voxelize(r, carry):
        for cc in range(CHUNK // 16):
            o = r * CHUNK + cc * 16
            vx = gbuf[0, pl.ds(o, 16)]
            vy = gbuf[1, pl.ds(o, 16)]
            vz = gbuf[2, pl.ds(o, 16)]
            ix = (vx / _DX).astype(jnp.int32)
            iy = (vy / _DX).astype(jnp.int32)
            iz = (vz / _DZ).astype(jnp.int32)
            kept = (
                (ix >= 0) & (ix < GRID)
                & (iy >= 0) & (iy < GRID)
                & (iz >= 0) & (iz < 1)
            )
            lin = ix * GRID + iy
            idxbuf[r, pl.ds(cc * 16, 16)] = jnp.where(kept, lin, DUMP)
        return carry

    lax.fori_loop(0, NCHUNK, _voxelize, 0)

    for p in range(NPASS):
        for z in range(ROWS_PER_TILE // ZROWS):
            pltpu.sync_copy(
                zbuf, acc.at[pl.ds(s * ROWS_PER_TILE + z * ZROWS, ZROWS)])
        plsc.subcore_barrier()

        def _group(m, carry, p=p):
            t0 = m * NBUF
            loads = []
            for k in range(NBUF):
                loads.append(pltpu.async_copy(
                    x_hbm.at[pl.ds(gbase + (t0 + k) * SLAB, SLAB),
                             pl.ds(p * CH, CH)],
                    xbuf.at[k], lsems[k]))
            scats = []
            for k in range(NBUF):
                loads[k].wait()
                for j in range(CPS):
                    scats.append(pltpu.async_copy(
                        xbuf.at[k, pl.ds(j * CHUNK, CHUNK)],
                        acc.at[idxbuf.at[(t0 + k) * CPS + j]],
                        ssem, add=True))
            for h in scats:
                h.wait()
            return carry

        lax.fori_loop(0, NGRP, _group, 0)
        plsc.subcore_barrier()

        pltpu.sync_copy(
            acc.at[pl.ds(s * ROWS_PER_TILE, ROWS_PER_TILE)],
            out_hbm.at[c, p, pl.ds(s * ROWS_PER_TILE, ROWS_PER_TILE)],
        )
        plsc.subcore_barrier()


_TBLK = 2560


def _tc_t_body(in_ref, out_ref):
    out_ref[...] = jnp.swapaxes(in_ref[...], -1, -2)


def _tc_transpose(xin):
    bp = B * NPASS
    return pl.pallas_call(
        _tc_t_body,
        grid=(bp, pl.cdiv(CELLS, _TBLK)),
        in_specs=[pl.BlockSpec((1, _TBLK, CH), lambda i, j: (i, j, 0))],
        out_specs=pl.BlockSpec((1, CH, _TBLK), lambda i, j: (i, 0, j)),
        out_shape=jax.ShapeDtypeStruct((bp, CH, CELLS), jnp.float32),
    )(xin)


def kernel(geom_feats, x):
    g = geom_feats.reshape(NPRIME, 3).T
    x2d = x.reshape(NPRIME, C)
    out = _bevpool_sc(g, x2d)
    outt = _tc_transpose(out.reshape(B * NPASS, CELLS, CH))
    return outt.reshape(B, C, GRID, GRID)
